# baseline (speedup 1.0000x reference)
import jax
import jax.numpy as jnp
from jax import lax
from jax.experimental import pallas as pl
from jax.experimental.pallas import tpu as pltpu
from jax.experimental.pallas import tpu_sc as plsc

_B = 65536
_SIN = 64
_SOUT = 256
_L = 16
_NC = 2
_NS = 16
_NW = _NC * _NS
_ROWS_PER_W = _B // _NW
_CHUNK = 128
_NCHUNK = _ROWS_PER_W // _CHUNK
_UNROLL = 4
_NBUF = 2


def _sc_body(y_hbm, no_hbm, out_hbm, no_v, xbuf, obuf, *sems):
    isems = sems[:2]
    osems = sems[2:]
    wid = lax.axis_index("s") * _NC + lax.axis_index("c")
    base = wid * _ROWS_PER_W

    pltpu.sync_copy(no_hbm, no_v)
    nov = [no_v[pl.ds(k * _L, _L)] for k in range(_SIN // _L)]

    zf = jnp.zeros((_L,), jnp.float32)

    def zero_body(r, c):
        for b in range(_NBUF):
            for j in range(_SOUT // _L):
                obuf[b, r, pl.ds(j * _L, _L)] = zf
        return c

    lax.fori_loop(0, _CHUNK, zero_body, 0)

    for b in range(2):
        pltpu.async_copy(
            y_hbm.at[pl.ds(base + b * _CHUNK, _CHUNK)], xbuf.at[b], isems[b]
        )

    for chunk in range(_NCHUNK):
        bi = chunk % 2
        bo = chunk % _NBUF
        r0 = base + chunk * _CHUNK
        pltpu.make_async_copy(
            y_hbm.at[pl.ds(r0, _CHUNK)], xbuf.at[bi], isems[bi]
        ).wait()

        if chunk >= _NBUF:
            pltpu.make_async_copy(
                obuf.at[bo], out_hbm.at[pl.ds(r0, _CHUNK)], osems[bo]
            ).wait()

        def row_body(i, cc, bi=bi, bo=bo):
            r = i * _UNROLL
            for u in range(_UNROLL):
                rs = jnp.full((_L,), r + u, jnp.int32)
                for k in range(_SIN // _L):
                    v = xbuf[bi, r + u, pl.ds(k * _L, _L)]
                    plsc.store_scatter(obuf.at[bo], [rs, nov[k]], v)
            return cc

        lax.fori_loop(0, _CHUNK // _UNROLL, row_body, 0)

        pltpu.async_copy(obuf.at[bo], out_hbm.at[pl.ds(r0, _CHUNK)], osems[bo])

        if chunk + 2 < _NCHUNK:
            pltpu.async_copy(
                y_hbm.at[pl.ds(r0 + 2 * _CHUNK, _CHUNK)], xbuf.at[bi], isems[bi]
            )

    for b in range(_NBUF):
        pltpu.make_async_copy(
            obuf.at[b], out_hbm.at[pl.ds(base, _CHUNK)], osems[b]
        ).wait()


def _make_sc_call():
    return pl.kernel(
        _sc_body,
        name="scatter_cols",
        out_type=jax.ShapeDtypeStruct((_B, _SOUT), jnp.float32),
        mesh=plsc.VectorSubcoreMesh(
            core_axis_name="c", subcore_axis_name="s", num_cores=_NC, num_subcores=_NS
        ),
        compiler_params=pltpu.CompilerParams(needs_layout_passes=False),
        scratch_types=(
            [
                pltpu.VMEM((_SIN,), jnp.int32),
                pltpu.VMEM((2, _CHUNK, _SIN), jnp.float32),
                pltpu.VMEM((_NBUF, _CHUNK, _SOUT), jnp.float32),
            ]
            + [pltpu.SemaphoreType.DMA] * (2 + _NBUF)
        ),
    )


_TR_BLK = 1024


def _tc_scale_transpose(xt_ref, w_ref, y_ref):
    y_ref[...] = (w_ref[...] * xt_ref[...]).T


def _tc_stage(xt, weights):
    return pl.pallas_call(
        _tc_scale_transpose,
        grid=(_B // _TR_BLK,),
        in_specs=[
            pl.BlockSpec((_SIN, _TR_BLK), lambda i: (0, i)),
            pl.BlockSpec((_SIN, 1), lambda i: (0, 0)),
        ],
        out_specs=pl.BlockSpec((_TR_BLK, _SIN), lambda i: (i, 0)),
        out_shape=jax.ShapeDtypeStruct((_B, _SIN), jnp.float32),
        name="scale_transpose",
    )(xt, weights.reshape(_SIN, 1))


@jax.jit
def kernel(x, weights, node_order):
    y = _tc_stage(x.T, weights)
    return _make_sc_call()(y, node_order)


# speedup vs baseline: 1.2528x; 1.2528x over previous
import jax
import jax.numpy as jnp
from jax import lax
from jax.experimental import pallas as pl
from jax.experimental.pallas import tpu as pltpu
from jax.experimental.pallas import tpu_sc as plsc

_B = 65536
_SIN = 64
_SOUT = 256
_L = 16
_NC = 2
_NS = 16
_NW = _NC * _NS
_ROWS_PER_W = _B // _NW
_CHUNK = 128
_NCHUNK = _ROWS_PER_W // _CHUNK
_UNROLL = 4
_NBUF = 2


def _sc_body(x_hbm, w_hbm, no_hbm, out_hbm, no_v, w_v, xbuf, obuf, *sems):
    isems = sems[:2]
    osems = sems[2:]
    wid = lax.axis_index("s") * _NC + lax.axis_index("c")
    base = wid * _ROWS_PER_W

    for b in range(2):
        pltpu.async_copy(
            x_hbm.at[pl.ds(base + b * _CHUNK, _CHUNK)], xbuf.at[b], isems[b]
        )

    pltpu.sync_copy(no_hbm, no_v)
    pltpu.sync_copy(w_hbm, w_v)
    nov = [no_v[pl.ds(k * _L, _L)] for k in range(_SIN // _L)]
    wv = [w_v[pl.ds(k * _L, _L)] for k in range(_SIN // _L)]

    zf = jnp.zeros((_L,), jnp.float32)

    def zero_body(r, c):
        for b in range(_NBUF):
            for j in range(_SOUT // _L):
                obuf[b, r, pl.ds(j * _L, _L)] = zf
        return c

    lax.fori_loop(0, _CHUNK, zero_body, 0)

    for chunk in range(_NCHUNK):
        bi = chunk % 2
        bo = chunk % _NBUF
        r0 = base + chunk * _CHUNK
        pltpu.make_async_copy(
            x_hbm.at[pl.ds(r0, _CHUNK)], xbuf.at[bi], isems[bi]
        ).wait()

        if chunk >= _NBUF:
            pltpu.make_async_copy(
                obuf.at[bo], out_hbm.at[pl.ds(r0, _CHUNK)], osems[bo]
            ).wait()

        def row_body(i, cc, bi=bi, bo=bo):
            r = i * _UNROLL
            for u in range(_UNROLL):
                rs = jnp.full((_L,), r + u, jnp.int32)
                for k in range(_SIN // _L):
                    v = xbuf[bi, r + u, pl.ds(k * _L, _L)] * wv[k]
                    plsc.store_scatter(obuf.at[bo], [rs, nov[k]], v)
            return cc

        lax.fori_loop(0, _CHUNK // _UNROLL, row_body, 0)

        pltpu.async_copy(obuf.at[bo], out_hbm.at[pl.ds(r0, _CHUNK)], osems[bo])

        if chunk + 2 < _NCHUNK:
            pltpu.async_copy(
                x_hbm.at[pl.ds(r0 + 2 * _CHUNK, _CHUNK)], xbuf.at[bi], isems[bi]
            )

    for b in range(_NBUF):
        pltpu.make_async_copy(
            obuf.at[b], out_hbm.at[pl.ds(base, _CHUNK)], osems[b]
        ).wait()


def _make_sc_call():
    return pl.kernel(
        _sc_body,
        name="scatter_cols",
        out_type=jax.ShapeDtypeStruct((_B, _SOUT), jnp.float32),
        mesh=plsc.VectorSubcoreMesh(
            core_axis_name="c", subcore_axis_name="s", num_cores=_NC, num_subcores=_NS
        ),
        compiler_params=pltpu.CompilerParams(needs_layout_passes=False),
        scratch_types=(
            [
                pltpu.VMEM((_SIN,), jnp.int32),
                pltpu.VMEM((_SIN,), jnp.float32),
                pltpu.VMEM((2, _CHUNK, _SIN), jnp.float32),
                pltpu.VMEM((_NBUF, _CHUNK, _SOUT), jnp.float32),
            ]
            + [pltpu.SemaphoreType.DMA] * (2 + _NBUF)
        ),
    )


@jax.jit
def kernel(x, weights, node_order):
    return _make_sc_call()(x, weights, node_order)


# speedup vs baseline: 1.2652x; 1.0099x over previous
import jax
import jax.numpy as jnp
from jax import lax
from jax.experimental import pallas as pl
from jax.experimental.pallas import tpu as pltpu
from jax.experimental.pallas import tpu_sc as plsc

_B = 65536
_SIN = 64
_SOUT = 256
_L = 16
_NC = 2
_NS = 16
_NW = _NC * _NS
_ROWS_PER_W = _B // _NW
_CHUNK = 64
_NCHUNK = _ROWS_PER_W // _CHUNK
_UNROLL = 4
_NBUF = 4


def _sc_body(x_hbm, w_hbm, no_hbm, out_hbm, no_v, w_v, xbuf, obuf, *sems):
    isems = sems[:2]
    osems = sems[2:]
    wid = lax.axis_index("s") * _NC + lax.axis_index("c")
    base = wid * _ROWS_PER_W

    for b in range(2):
        pltpu.async_copy(
            x_hbm.at[pl.ds(base + b * _CHUNK, _CHUNK)], xbuf.at[b], isems[b]
        )

    pltpu.sync_copy(no_hbm, no_v)
    pltpu.sync_copy(w_hbm, w_v)
    nov = [no_v[pl.ds(k * _L, _L)] for k in range(_SIN // _L)]
    wv = [w_v[pl.ds(k * _L, _L)] for k in range(_SIN // _L)]

    zf = jnp.zeros((_L,), jnp.float32)

    def zero_body(r, c):
        for b in range(_NBUF):
            for j in range(_SOUT // _L):
                obuf[b, r, pl.ds(j * _L, _L)] = zf
        return c

    lax.fori_loop(0, _CHUNK, zero_body, 0)

    _PH = 4

    def outer(t, carry):
        for ph in range(_PH):
            chunk = _PH * t + ph
            bi = ph % 2
            bo = ph % _NBUF
            r0 = base + chunk * _CHUNK
            pltpu.make_async_copy(
                x_hbm.at[pl.ds(r0, _CHUNK)], xbuf.at[bi], isems[bi]
            ).wait()

            @pl.when(chunk >= _NBUF)
            def _wait_out():
                pltpu.make_async_copy(
                    obuf.at[bo], out_hbm.at[pl.ds(r0, _CHUNK)], osems[bo]
                ).wait()

            def row_body(i, cc, bi=bi, bo=bo):
                r = i * _UNROLL
                for u in range(_UNROLL):
                    rs = jnp.full((_L,), r + u, jnp.int32)
                    for k in range(_SIN // _L):
                        v = xbuf[bi, r + u, pl.ds(k * _L, _L)] * wv[k]
                        plsc.store_scatter(obuf.at[bo], [rs, nov[k]], v)
                return cc

            lax.fori_loop(0, _CHUNK // _UNROLL, row_body, 0)

            pltpu.async_copy(obuf.at[bo], out_hbm.at[pl.ds(r0, _CHUNK)], osems[bo])

            @pl.when(chunk + 2 < _NCHUNK)
            def _next_in():
                pltpu.async_copy(
                    x_hbm.at[pl.ds(r0 + 2 * _CHUNK, _CHUNK)], xbuf.at[bi], isems[bi]
                )

        return carry

    lax.fori_loop(0, _NCHUNK // _PH, outer, 0)

    for b in range(_NBUF):
        pltpu.make_async_copy(
            obuf.at[b], out_hbm.at[pl.ds(base, _CHUNK)], osems[b]
        ).wait()


def _make_sc_call():
    return pl.kernel(
        _sc_body,
        name="scatter_cols",
        out_type=jax.ShapeDtypeStruct((_B, _SOUT), jnp.float32),
        mesh=plsc.VectorSubcoreMesh(
            core_axis_name="c", subcore_axis_name="s", num_cores=_NC, num_subcores=_NS
        ),
        compiler_params=pltpu.CompilerParams(needs_layout_passes=False),
        scratch_types=(
            [
                pltpu.VMEM((_SIN,), jnp.int32),
                pltpu.VMEM((_SIN,), jnp.float32),
                pltpu.VMEM((2, _CHUNK, _SIN), jnp.float32),
                pltpu.VMEM((_NBUF, _CHUNK, _SOUT), jnp.float32),
            ]
            + [pltpu.SemaphoreType.DMA] * (2 + _NBUF)
        ),
    )


@jax.jit
def kernel(x, weights, node_order):
    return _make_sc_call()(x, weights, node_order)


# speedup vs baseline: 1.3124x; 1.0373x over previous
import jax
import jax.numpy as jnp
from jax import lax
from jax.experimental import pallas as pl
from jax.experimental.pallas import tpu as pltpu
from jax.experimental.pallas import tpu_sc as plsc

_B = 65536
_SIN = 64
_SOUT = 256
_L = 16
_NC = 2
_NS = 16
_NW = _NC * _NS
_ROWS_PER_W = _B // _NW
_CHUNK = 128
_NCHUNK = _ROWS_PER_W // _CHUNK
_UNROLL = 4
_NBUF = 2


def _sc_body(x_hbm, w_hbm, no_hbm, out_hbm, no_v, w_v, xbuf, obuf, *sems):
    isems = sems[:2]
    osems = sems[2:]
    wid = lax.axis_index("s") * _NC + lax.axis_index("c")
    base = wid * _ROWS_PER_W

    for b in range(2):
        pltpu.async_copy(
            x_hbm.at[pl.ds(base + b * _CHUNK, _CHUNK)], xbuf.at[b], isems[b]
        )

    pltpu.sync_copy(no_hbm, no_v)
    pltpu.sync_copy(w_hbm, w_v)
    nov = [no_v[pl.ds(k * _L, _L)] for k in range(_SIN // _L)]
    wv = [w_v[pl.ds(k * _L, _L)] for k in range(_SIN // _L)]

    zf = jnp.zeros((_L,), jnp.float32)

    def zero_body(r, c):
        for b in range(_NBUF):
            for j in range(_SOUT // _L):
                obuf[b, r, pl.ds(j * _L, _L)] = zf
        return c

    lax.fori_loop(0, _CHUNK, zero_body, 0)

    _PH = 2

    def outer(t, carry):
        for ph in range(_PH):
            chunk = _PH * t + ph
            bi = ph % 2
            bo = ph % _NBUF
            r0 = base + chunk * _CHUNK
            pltpu.make_async_copy(
                x_hbm.at[pl.ds(r0, _CHUNK)], xbuf.at[bi], isems[bi]
            ).wait()

            @pl.when(chunk >= _NBUF)
            def _wait_out():
                pltpu.make_async_copy(
                    obuf.at[bo], out_hbm.at[pl.ds(r0, _CHUNK)], osems[bo]
                ).wait()

            def row_body(i, cc, bi=bi, bo=bo):
                r = i * _UNROLL
                for u in range(_UNROLL):
                    rs = jnp.full((_L,), r + u, jnp.int32)
                    for k in range(_SIN // _L):
                        v = xbuf[bi, r + u, pl.ds(k * _L, _L)] * wv[k]
                        plsc.store_scatter(obuf.at[bo], [rs, nov[k]], v)
                return cc

            lax.fori_loop(0, _CHUNK // _UNROLL, row_body, 0)

            pltpu.async_copy(obuf.at[bo], out_hbm.at[pl.ds(r0, _CHUNK)], osems[bo])

            @pl.when(chunk + 2 < _NCHUNK)
            def _next_in():
                pltpu.async_copy(
                    x_hbm.at[pl.ds(r0 + 2 * _CHUNK, _CHUNK)], xbuf.at[bi], isems[bi]
                )

        return carry

    lax.fori_loop(0, _NCHUNK // _PH, outer, 0)

    for b in range(_NBUF):
        pltpu.make_async_copy(
            obuf.at[b], out_hbm.at[pl.ds(base, _CHUNK)], osems[b]
        ).wait()


def _make_sc_call():
    return pl.kernel(
        _sc_body,
        name="scatter_cols",
        out_type=jax.ShapeDtypeStruct((_B, _SOUT), jnp.float32),
        mesh=plsc.VectorSubcoreMesh(
            core_axis_name="c", subcore_axis_name="s", num_cores=_NC, num_subcores=_NS
        ),
        compiler_params=pltpu.CompilerParams(needs_layout_passes=False),
        scratch_types=(
            [
                pltpu.VMEM((_SIN,), jnp.int32),
                pltpu.VMEM((_SIN,), jnp.float32),
                pltpu.VMEM((2, _CHUNK, _SIN), jnp.float32),
                pltpu.VMEM((_NBUF, _CHUNK, _SOUT), jnp.float32),
            ]
            + [pltpu.SemaphoreType.DMA] * (2 + _NBUF)
        ),
    )


@jax.jit
def kernel(x, weights, node_order):
    return _make_sc_call()(x, weights, node_order)
